# baseline (speedup 1.0000x reference)
import functools

import jax
import jax.numpy as jnp
import numpy as np
from jax import lax
from jax.experimental import pallas as pl
from jax.experimental.pallas import tpu as pltpu
from jax.experimental.pallas import tpu_sc as plsc

_B = 4096
_H = 200
_D = 32
_NW = 32
_BPW = _B // _NW
_C0 = 128
_C1 = _H - _C0
_NBUF = 4


def _pool_body(x_hbm, emb_hbm, out_hbm, idx_v, rows_v, acc_v, *sems):
    c = lax.axis_index("c")
    s = lax.axis_index("s")
    wid = s * 2 + c
    base = wid * _BPW
    pltpu.sync_copy(x_hbm.at[pl.ds(base, _BPW)], idx_v)

    def issue(b, buf):
        pltpu.async_copy(emb_hbm.at[idx_v.at[b, pl.ds(0, _C0)]],
                         rows_v.at[buf, pl.ds(0, _C0)], sems[buf])
        pltpu.async_copy(emb_hbm.at[idx_v.at[b, pl.ds(_C0, _C1)]],
                         rows_v.at[buf, pl.ds(_C0, _C1)], sems[buf])

    def drain(b, buf):
        pltpu.make_async_copy(emb_hbm.at[idx_v.at[b, pl.ds(0, _C0)]],
                              rows_v.at[buf, pl.ds(0, _C0)], sems[buf]).wait()
        pltpu.make_async_copy(emb_hbm.at[idx_v.at[b, pl.ds(_C0, _C1)]],
                              rows_v.at[buf, pl.ds(_C0, _C1)], sems[buf]).wait()

    for p in range(_NBUF):
        issue(p, p)

    hi_mask = jnp.full((16,), -65536, dtype=jnp.int32)

    def one_group(t, carry):
        for p in range(_NBUF):
            b = _NBUF * t + p
            drain(b, p)

            def red(i, acc):
                new = list(acc)
                for k in range(2):
                    j = 2 * i + k
                    v = plsc.bitcast(rows_v[p, j, 0:32], jnp.int32)
                    lo = plsc.bitcast(v << 16, jnp.float32)
                    hi = plsc.bitcast(v & hi_mask, jnp.float32)
                    new[2 * k] = new[2 * k] + lo
                    new[2 * k + 1] = new[2 * k + 1] + hi
                return tuple(new)

            z = jnp.zeros((16,), jnp.float32)
            acc = lax.fori_loop(0, _H // 2, red, (z,) * 4, unroll=8)
            acc_v[b, 0:16] = acc[0] + acc[2]
            acc_v[b, 16:32] = acc[1] + acc[3]

            @pl.when(b + _NBUF < _BPW)
            def _():
                issue(b + _NBUF, p)
        return carry

    lax.fori_loop(0, _BPW // _NBUF, one_group, 0)
    pltpu.sync_copy(acc_v, out_hbm.at[pl.ds(base, _BPW)])


_pool = functools.partial(
    pl.kernel,
    out_type=jax.ShapeDtypeStruct((_B, _D), jnp.float32),
    mesh=plsc.VectorSubcoreMesh(core_axis_name="c", subcore_axis_name="s"),
    scratch_types=[
        pltpu.VMEM((_BPW, _H), jnp.int32),
        pltpu.VMEM((_NBUF, _H, _D), jnp.bfloat16),
        pltpu.VMEM((_BPW, _D), jnp.float32),
    ] + [pltpu.SemaphoreType.DMA] * _NBUF,
    compiler_params=pltpu.CompilerParams(use_tc_tiling_on_sc=False,
                                         needs_layout_passes=False),
)(_pool_body)

_PERM = np.concatenate([np.arange(0, _D, 2), np.arange(1, _D, 2)])


def _mlp_body(p_ref, w1_ref, b1_ref, w2_ref, b2_ref, o_ref):
    h = p_ref[...] * (1.0 / _H)
    h = jnp.dot(h, w1_ref[...], preferred_element_type=jnp.float32) + b1_ref[...]
    h = jnp.maximum(h, 0.0)
    o_ref[...] = jnp.dot(h, w2_ref[...], preferred_element_type=jnp.float32) + b2_ref[...]


def kernel(x, emb, W1, b1, W2, b2):
    emb_bf = emb.astype(jnp.bfloat16)
    pooled = _pool(x, emb_bf)
    w1p = W1[_PERM, :]
    w2p = jnp.zeros((_D, 128), jnp.float32).at[:, :10].set(W2)
    b2p = jnp.zeros((1, 128), jnp.float32).at[:, :10].set(b2)
    out = pl.pallas_call(
        _mlp_body,
        out_shape=jax.ShapeDtypeStruct((_B, 128), jnp.float32),
    )(pooled, w1p, b1.reshape(1, _D), w2p, b2p)
    return out[:, :10]


# speedup vs baseline: 1.1668x; 1.1668x over previous
import functools

import jax
import jax.numpy as jnp
import numpy as np
from jax import lax
from jax.experimental import pallas as pl
from jax.experimental.pallas import tpu as pltpu
from jax.experimental.pallas import tpu_sc as plsc

_B = 4096
_H = 200
_D = 32
_NW = 32
_BPW = _B // _NW
_C0 = 128
_C1 = _H - _C0
_NBUF = 4


def _pool_body(x_hbm, emb_hbm, out_hbm, idx_v, rows_v, acc_v, *sems):
    c = lax.axis_index("c")
    s = lax.axis_index("s")
    wid = s * 2 + c
    base = wid * _BPW
    pltpu.sync_copy(x_hbm.at[pl.ds(base, _BPW)], idx_v)

    def issue(b, buf):
        pltpu.async_copy(emb_hbm.at[idx_v.at[b, pl.ds(0, _C0)]],
                         rows_v.at[buf, pl.ds(0, _C0)], sems[buf])
        pltpu.async_copy(emb_hbm.at[idx_v.at[b, pl.ds(_C0, _C1)]],
                         rows_v.at[buf, pl.ds(_C0, _C1)], sems[buf])

    def drain(b, buf):
        pltpu.make_async_copy(emb_hbm.at[idx_v.at[b, pl.ds(0, _C0)]],
                              rows_v.at[buf, pl.ds(0, _C0)], sems[buf]).wait()
        pltpu.make_async_copy(emb_hbm.at[idx_v.at[b, pl.ds(_C0, _C1)]],
                              rows_v.at[buf, pl.ds(_C0, _C1)], sems[buf]).wait()

    for p in range(_NBUF):
        issue(p, p)

    hi_mask = jnp.full((16,), -65536, dtype=jnp.int32)

    def one_group(t, carry):
        for p in range(_NBUF):
            b = _NBUF * t + p
            drain(b, p)

            def red(i, acc):
                new = list(acc)
                for k in range(2):
                    j = 2 * i + k
                    new[2 * k] = new[2 * k] + rows_v[p, j, 0:16]
                    new[2 * k + 1] = new[2 * k + 1] + rows_v[p, j, 16:32]
                return tuple(new)

            z = jnp.zeros((16,), jnp.float32)
            acc = lax.fori_loop(0, _H // 2, red, (z,) * 4, unroll=8)
            acc_v[b, 0:16] = acc[0] + acc[2]
            acc_v[b, 16:32] = acc[1] + acc[3]

            @pl.when(b + _NBUF < _BPW)
            def _():
                issue(b + _NBUF, p)
        return carry

    lax.fori_loop(0, _BPW // _NBUF, one_group, 0)
    pltpu.sync_copy(acc_v, out_hbm.at[pl.ds(base, _BPW)])


_pool = functools.partial(
    pl.kernel,
    out_type=jax.ShapeDtypeStruct((_B, _D), jnp.float32),
    mesh=plsc.VectorSubcoreMesh(core_axis_name="c", subcore_axis_name="s"),
    scratch_types=[
        pltpu.VMEM((_BPW, _H), jnp.int32),
        pltpu.VMEM((_NBUF, _H, _D), jnp.float32),
        pltpu.VMEM((_BPW, _D), jnp.float32),
    ] + [pltpu.SemaphoreType.DMA] * _NBUF,
    compiler_params=pltpu.CompilerParams(use_tc_tiling_on_sc=False,
                                         needs_layout_passes=False),
)(_pool_body)

_PERM = np.concatenate([np.arange(0, _D, 2), np.arange(1, _D, 2)])


def _mlp_body(p_ref, w1_ref, b1_ref, w2_ref, b2_ref, o_ref):
    h = p_ref[...] * (1.0 / _H)
    h = jnp.dot(h, w1_ref[...], preferred_element_type=jnp.float32) + b1_ref[...]
    h = jnp.maximum(h, 0.0)
    o_ref[...] = jnp.dot(h, w2_ref[...], preferred_element_type=jnp.float32) + b2_ref[...]


def kernel(x, emb, W1, b1, W2, b2):
    pooled = _pool(x, emb)
    w1p = W1
    w2p = jnp.zeros((_D, 128), jnp.float32).at[:, :10].set(W2)
    b2p = jnp.zeros((1, 128), jnp.float32).at[:, :10].set(b2)
    out = pl.pallas_call(
        _mlp_body,
        out_shape=jax.ShapeDtypeStruct((_B, 128), jnp.float32),
    )(pooled, w1p, b1.reshape(1, _D), w2p, b2p)
    return out[:, :10]
